# baseline (speedup 1.0000x reference)
import jax
import jax.numpy as jnp
from jax import lax
from jax.experimental import pallas as pl
from jax.experimental.pallas import tpu as pltpu
from jax.experimental.pallas import tpu_sc as plsc

LR_ = 0.01
WD_ = 1e-05
N_ITEMS = 1000000
N_USERS = 100000
DIM = 16
B_ = 16384

NCORE = 2
NSUB = 16
PER_TILE = B_ // NSUB
CHUNK = 128
NCHUNK = PER_TILE // CHUNK

N_OUT = N_ITEMS + N_USERS
CB = 8192


def _combine_body(tin_ref, d_ref, out_ref):
    out_ref[...] = tin_ref[...] * (1.0 - WD_) - d_ref[...]


def _combine(tin, delta_t):
    n = tin.shape[1]
    grid = (n + CB - 1) // CB
    return pl.pallas_call(
        _combine_body,
        grid=(grid,),
        in_specs=[
            pl.BlockSpec((16, CB), lambda i: (0, i)),
            pl.BlockSpec((16, CB), lambda i: (0, i)),
        ],
        out_specs=pl.BlockSpec((16, CB), lambda i: (0, i)),
        out_shape=jax.ShapeDtypeStruct((16, n), jnp.float32),
    )(tin, delta_t)


def _sc_body(dti_hbm, dtu_hbm, idxl_hbm, occ_hbm, grads_hbm, zeros2_hbm,
             zeros1_hbm, ones1_hbm, neg1_hbm,
             slot_sp, cnt_sp, accum_sp,
             idxl_v, occ_v, rep_v, cnt_v, inv_v, ones_v,
             grad_v, svals_v, gidx_v, nidx_v, nslot_v, cl_v, kidx_v):
    c = lax.axis_index("c")
    s = lax.axis_index("s")
    iota16 = lax.iota(jnp.int32, 16)
    occ_base = c * B_ + s * PER_TILE
    row8 = s * NCHUNK

    pltpu.sync_copy(idxl_hbm.at[pl.ds(row8 + c * (B_ // CHUNK), NCHUNK)], idxl_v)
    pltpu.sync_copy(occ_hbm.at[pl.ds(row8, NCHUNK)], occ_v)
    pltpu.sync_copy(grads_hbm.at[pl.ds(occ_base, PER_TILE)], grad_v)
    pltpu.sync_copy(ones1_hbm, ones_v)

    pltpu.sync_copy(zeros2_hbm, accum_sp.at[pl.ds(s * PER_TILE, PER_TILE)])
    pltpu.sync_copy(zeros1_hbm, cnt_sp.at[pl.ds(s * PER_TILE, PER_TILE)])

    @pl.when(s == 0)
    def _():
        pltpu.sync_copy(neg1_hbm, slot_sp)
        pltpu.sync_copy(zeros2_hbm.at[pl.ds(0, 8)], accum_sp.at[pl.ds(B_, 8)])

    plsc.subcore_barrier()

    for j in range(NCHUNK):
        pltpu.sync_copy(occ_v.at[j], slot_sp.at[idxl_v.at[j]])

    plsc.subcore_barrier()

    for j in range(NCHUNK):
        pltpu.sync_copy(slot_sp.at[idxl_v.at[j]], rep_v.at[j])

    for j in range(NCHUNK):
        pltpu.sync_copy(ones_v, cnt_sp.at[rep_v.at[j]], add=True)

    plsc.subcore_barrier()

    for j in range(NCHUNK):
        pltpu.sync_copy(cnt_sp.at[rep_v.at[j]], cnt_v.at[j])
    for j in range(NCHUNK):
        for k in range(CHUNK // 16):
            inv_v[j, pl.ds(k * 16, 16)] = LR_ / cnt_v[j, pl.ds(k * 16, 16)]

    for j in range(NCHUNK):
        @pl.loop(0, CHUNK // 16)
        def _(k, j=j):
            iv = inv_v[j, pl.ds(k * 16, 16)]
            base_i = j * CHUNK + k * 16
            for r in range(16):
                grad_v[base_i + r, :] = iv[r] * grad_v[base_i + r, :]

    for j in range(NCHUNK):
        pltpu.sync_copy(grad_v.at[pl.ds(j * CHUNK, CHUNK)],
                        accum_sp.at[rep_v.at[j]], add=True)

    plsc.subcore_barrier()

    for j in range(NCHUNK):
        for r in range(8):
            gidx_v[pl.ds(r * 16, 16)] = lax.shift_right_logical(
                idxl_v[j, pl.ds(r * 16, 16)], 3)

        @pl.loop(0, 8)
        def _(q, j=j):
            for r in range(8):
                pvec = q * 128 + r * 16 + iota16
                ovec = lax.shift_right_logical(pvec, 3)
                jjvec = lax.bitwise_and(pvec, 7)
                il = plsc.load_gather(
                    idxl_v, [jnp.full((16,), j, jnp.int32), ovec])
                nidx_v[pl.ds(r * 16, 16)] = (
                    lax.bitwise_and(il, jnp.int32(-8)) + jjvec)
            pltpu.sync_copy(slot_sp.at[nidx_v], nslot_v)
            for r in range(8):
                ns = nslot_v[pl.ds(r * 16, 16)]
                cl_v[pl.ds(r * 16, 16)] = jnp.where(ns < 0, B_, ns)
            pltpu.sync_copy(accum_sp.at[cl_v], grad_v.at[pl.ds(q * 128, 128)])

        @pl.loop(0, PER_TILE)
        def _(p, j=j):
            row = jnp.full((16,), lax.shift_right_logical(p, 3), jnp.int32)
            col = jnp.full((16,), lax.bitwise_and(p, 7), jnp.int32)
            plsc.store_scatter(svals_v, [iota16, row, col], grad_v[p, :])

        @pl.when(c == 0)
        def _(j=j):
            @pl.loop(0, 16)
            def _(k):
                for r in range(8):
                    kidx_v[pl.ds(r * 16, 16)] = (
                        gidx_v[pl.ds(r * 16, 16)] + k * (N_ITEMS // 8))
                pltpu.sync_copy(svals_v.at[k], dti_hbm.at[kidx_v])

        @pl.when(c == 1)
        def _(j=j):
            @pl.loop(0, 16)
            def _(k):
                for r in range(8):
                    kidx_v[pl.ds(r * 16, 16)] = (
                        gidx_v[pl.ds(r * 16, 16)] + k * (N_USERS // 8))
                pltpu.sync_copy(svals_v.at[k], dtu_hbm.at[kidx_v])


_sc_fixup = pl.kernel(
    _sc_body,
    out_type=(),
    mesh=plsc.VectorSubcoreMesh(core_axis_name="c", subcore_axis_name="s"),
    compiler_params=pltpu.CompilerParams(use_tc_tiling_on_sc=False,
                                         needs_layout_passes=False),
    scratch_types=[
        pltpu.VMEM_SHARED((N_ITEMS,), jnp.int32),
        pltpu.VMEM_SHARED((B_,), jnp.float32),
        pltpu.VMEM_SHARED((B_ + 8, DIM), jnp.float32),
        pltpu.VMEM((NCHUNK, CHUNK), jnp.int32),
        pltpu.VMEM((NCHUNK, CHUNK), jnp.int32),
        pltpu.VMEM((NCHUNK, CHUNK), jnp.int32),
        pltpu.VMEM((NCHUNK, CHUNK), jnp.float32),
        pltpu.VMEM((NCHUNK, CHUNK), jnp.float32),
        pltpu.VMEM((CHUNK,), jnp.float32),
        pltpu.VMEM((PER_TILE, DIM), jnp.float32),
        pltpu.VMEM((16, CHUNK, 8), jnp.float32),
        pltpu.VMEM((CHUNK,), jnp.int32),
        pltpu.VMEM((CHUNK,), jnp.int32),
        pltpu.VMEM((CHUNK,), jnp.int32),
        pltpu.VMEM((CHUNK,), jnp.int32),
        pltpu.VMEM((CHUNK,), jnp.int32),
    ],
)


def kernel(item_emb, user_emb, item_grad, user_grad, returned_items,
           returned_users):
    ri = returned_items.astype(jnp.int32)
    ru = returned_users.astype(jnp.int32)
    idx_l = jnp.concatenate([ri, ru]).reshape(2 * B_ // CHUNK, CHUNK)
    occ = jnp.arange(B_, dtype=jnp.int32).reshape(B_ // CHUNK, CHUNK)
    grads = jnp.concatenate([item_grad, user_grad], axis=0)
    zeros2 = jnp.zeros((PER_TILE, DIM), jnp.float32)
    zeros1 = jnp.zeros((PER_TILE,), jnp.float32)
    ones1 = jnp.ones((CHUNK,), jnp.float32)

    neg1 = jnp.full((N_ITEMS,), -1, jnp.int32)
    dti_ref = jax.new_ref(jnp.zeros((16 * N_ITEMS // 8, 8), jnp.float32))
    dtu_ref = jax.new_ref(jnp.zeros((16 * N_USERS // 8, 8), jnp.float32))
    _sc_fixup(dti_ref, dtu_ref, idx_l, occ, grads, zeros2, zeros1, ones1, neg1)
    out_i = _combine(item_emb.T, dti_ref[...].reshape(16, N_ITEMS))
    out_u = _combine(user_emb.T, dtu_ref[...].reshape(16, N_USERS))
    return jnp.concatenate([out_i.T, out_u.T], axis=0)


# speedup vs baseline: 2.8709x; 2.8709x over previous
import jax
import jax.numpy as jnp
from jax import lax
from jax.experimental import pallas as pl
from jax.experimental.pallas import tpu as pltpu
from jax.experimental.pallas import tpu_sc as plsc

LR_ = 0.01
WD_ = 1e-05
N_ITEMS = 1000000
N_USERS = 100000
DIM = 16
B_ = 16384

NCORE = 2
NSUB = 16
PER_TILE = B_ // NSUB
CHUNK = 128
NCHUNK = PER_TILE // CHUNK

N_OUT = N_ITEMS + N_USERS
CB = 8192


def _combine_body(tin_ref, d_ref, out_ref):
    out_ref[...] = tin_ref[...] * (1.0 - WD_) - d_ref[...]


def _combine(tin, delta_t):
    n = tin.shape[1]
    grid = (n + CB - 1) // CB
    return pl.pallas_call(
        _combine_body,
        grid=(grid,),
        in_specs=[
            pl.BlockSpec((16, CB), lambda i: (0, i)),
            pl.BlockSpec((16, CB), lambda i: (0, i)),
        ],
        out_specs=pl.BlockSpec((16, CB), lambda i: (0, i)),
        out_shape=jax.ShapeDtypeStruct((16, n), jnp.float32),
    )(tin, delta_t)


def _sc_body(di_hbm, du_hbm, idxl_hbm, occ_hbm, grads_hbm, zeros2_hbm,
             zeros1_hbm, ones1_hbm,
             slot_sp, cnt_sp, accum_sp,
             idxl_v, occ_v, rep_v, cnt_v, inv_v, ones_v,
             grad_v):
    c = lax.axis_index("c")
    s = lax.axis_index("s")
    occ_base = c * B_ + s * PER_TILE
    row8 = s * NCHUNK

    pltpu.sync_copy(idxl_hbm.at[pl.ds(row8 + c * (B_ // CHUNK), NCHUNK)], idxl_v)
    pltpu.sync_copy(occ_hbm.at[pl.ds(row8, NCHUNK)], occ_v)
    pltpu.sync_copy(grads_hbm.at[pl.ds(occ_base, PER_TILE)], grad_v)
    pltpu.sync_copy(ones1_hbm, ones_v)

    pltpu.sync_copy(zeros2_hbm, accum_sp.at[pl.ds(s * PER_TILE, PER_TILE)])
    pltpu.sync_copy(zeros1_hbm, cnt_sp.at[pl.ds(s * PER_TILE, PER_TILE)])

    for j in range(NCHUNK):
        pltpu.sync_copy(occ_v.at[j], slot_sp.at[idxl_v.at[j]])

    plsc.subcore_barrier()

    for j in range(NCHUNK):
        pltpu.sync_copy(slot_sp.at[idxl_v.at[j]], rep_v.at[j])

    for j in range(NCHUNK):
        pltpu.sync_copy(ones_v, cnt_sp.at[rep_v.at[j]], add=True)

    plsc.subcore_barrier()

    for j in range(NCHUNK):
        pltpu.sync_copy(cnt_sp.at[rep_v.at[j]], cnt_v.at[j])
    for j in range(NCHUNK):
        for k in range(CHUNK // 16):
            inv_v[j, pl.ds(k * 16, 16)] = LR_ / cnt_v[j, pl.ds(k * 16, 16)]

    for j in range(NCHUNK):
        @pl.loop(0, CHUNK // 16)
        def _(k, j=j):
            iv = inv_v[j, pl.ds(k * 16, 16)]
            base_i = j * CHUNK + k * 16
            for r in range(16):
                grad_v[base_i + r, :] = iv[r] * grad_v[base_i + r, :]

    for j in range(NCHUNK):
        pltpu.sync_copy(grad_v.at[pl.ds(j * CHUNK, CHUNK)],
                        accum_sp.at[rep_v.at[j]], add=True)

    plsc.subcore_barrier()

    for j in range(NCHUNK):
        pltpu.sync_copy(accum_sp.at[rep_v.at[j]],
                        grad_v.at[pl.ds(j * CHUNK, CHUNK)])

    @pl.when(c == 0)
    def _():
        for j in range(NCHUNK):
            pltpu.sync_copy(grad_v.at[pl.ds(j * CHUNK, CHUNK)],
                            di_hbm.at[idxl_v.at[j]])

    @pl.when(c == 1)
    def _():
        for j in range(NCHUNK):
            pltpu.sync_copy(grad_v.at[pl.ds(j * CHUNK, CHUNK)],
                            du_hbm.at[idxl_v.at[j]])


_sc_fixup = pl.kernel(
    _sc_body,
    out_type=(),
    mesh=plsc.VectorSubcoreMesh(core_axis_name="c", subcore_axis_name="s"),
    compiler_params=pltpu.CompilerParams(use_tc_tiling_on_sc=False),
    scratch_types=[
        pltpu.VMEM_SHARED((N_ITEMS,), jnp.int32),
        pltpu.VMEM_SHARED((B_,), jnp.float32),
        pltpu.VMEM_SHARED((B_, DIM), jnp.float32),
        pltpu.VMEM((NCHUNK, CHUNK), jnp.int32),
        pltpu.VMEM((NCHUNK, CHUNK), jnp.int32),
        pltpu.VMEM((NCHUNK, CHUNK), jnp.int32),
        pltpu.VMEM((NCHUNK, CHUNK), jnp.float32),
        pltpu.VMEM((NCHUNK, CHUNK), jnp.float32),
        pltpu.VMEM((CHUNK,), jnp.float32),
        pltpu.VMEM((PER_TILE, DIM), jnp.float32),
    ],
)


def kernel(item_emb, user_emb, item_grad, user_grad, returned_items,
           returned_users):
    ri = returned_items.astype(jnp.int32)
    ru = returned_users.astype(jnp.int32)
    idx_l = jnp.concatenate([ri, ru]).reshape(2 * B_ // CHUNK, CHUNK)
    occ = jnp.arange(B_, dtype=jnp.int32).reshape(B_ // CHUNK, CHUNK)
    grads = jnp.concatenate([item_grad, user_grad], axis=0)
    zeros2 = jnp.zeros((PER_TILE, DIM), jnp.float32)
    zeros1 = jnp.zeros((PER_TILE,), jnp.float32)
    ones1 = jnp.ones((CHUNK,), jnp.float32)

    di_ref = jax.new_ref(jnp.zeros((N_ITEMS, DIM), jnp.float32))
    du_ref = jax.new_ref(jnp.zeros((N_USERS, DIM), jnp.float32))
    _sc_fixup(di_ref, du_ref, idx_l, occ, grads, zeros2, zeros1, ones1)
    out_i = _combine(item_emb.T, di_ref[...].T)
    out_u = _combine(user_emb.T, du_ref[...].T)
    return jnp.concatenate([out_i.T, out_u.T], axis=0)


# speedup vs baseline: 3.0378x; 1.0581x over previous
import jax
import jax.numpy as jnp
from jax import lax
from jax.experimental import pallas as pl
from jax.experimental.pallas import tpu as pltpu
from jax.experimental.pallas import tpu_sc as plsc

LR_ = 0.01
WD_ = 1e-05
N_ITEMS = 1000000
N_USERS = 100000
DIM = 16
B_ = 16384

NCORE = 2
NSUB = 16
PER_TILE = B_ // NSUB
CHUNK = 128
NCHUNK = PER_TILE // CHUNK

N_OUT = N_ITEMS + N_USERS
CB = 32768


def _combine_body(tin_ref, d_ref, out_ref):
    out_ref[...] = tin_ref[...] * (1.0 - WD_) - d_ref[...]


def _combine(tin, delta_t):
    n = tin.shape[1]
    grid = (n + CB - 1) // CB
    return pl.pallas_call(
        _combine_body,
        grid=(grid,),
        in_specs=[
            pl.BlockSpec((16, CB), lambda i: (0, i)),
            pl.BlockSpec((16, CB), lambda i: (0, i)),
        ],
        out_specs=pl.BlockSpec((16, CB), lambda i: (0, i)),
        out_shape=jax.ShapeDtypeStruct((16, n), jnp.float32),
    )(tin, delta_t)


def _sc_body(di_hbm, du_hbm, idxl_hbm, occ_hbm, grads_hbm, zeros2_hbm,
             zeros1_hbm, ones1_hbm,
             slot_sp, cnt_sp, accum_sp,
             idxl_v, occ_v, rep_v, cnt_v, inv_v, ones_v,
             grad_v):
    c = lax.axis_index("c")
    s = lax.axis_index("s")
    occ_base = c * B_ + s * PER_TILE
    row8 = s * NCHUNK

    pltpu.sync_copy(idxl_hbm.at[pl.ds(row8 + c * (B_ // CHUNK), NCHUNK)], idxl_v)
    pltpu.sync_copy(occ_hbm.at[pl.ds(row8, NCHUNK)], occ_v)
    pltpu.sync_copy(grads_hbm.at[pl.ds(occ_base, PER_TILE)], grad_v)
    pltpu.sync_copy(ones1_hbm, ones_v)

    pltpu.sync_copy(zeros2_hbm, accum_sp.at[pl.ds(s * PER_TILE, PER_TILE)])
    pltpu.sync_copy(zeros1_hbm, cnt_sp.at[pl.ds(s * PER_TILE, PER_TILE)])

    for j in range(NCHUNK):
        pltpu.sync_copy(occ_v.at[j], slot_sp.at[idxl_v.at[j]])

    plsc.subcore_barrier()

    for j in range(NCHUNK):
        pltpu.sync_copy(slot_sp.at[idxl_v.at[j]], rep_v.at[j])

    for j in range(NCHUNK):
        pltpu.sync_copy(ones_v, cnt_sp.at[rep_v.at[j]], add=True)

    plsc.subcore_barrier()

    for j in range(NCHUNK):
        pltpu.sync_copy(cnt_sp.at[rep_v.at[j]], cnt_v.at[j])
    for j in range(NCHUNK):
        for k in range(CHUNK // 16):
            inv_v[j, pl.ds(k * 16, 16)] = LR_ / cnt_v[j, pl.ds(k * 16, 16)]

    for j in range(NCHUNK):
        @pl.loop(0, CHUNK // 16)
        def _(k, j=j):
            iv = inv_v[j, pl.ds(k * 16, 16)]
            base_i = j * CHUNK + k * 16
            for r in range(16):
                grad_v[base_i + r, :] = iv[r] * grad_v[base_i + r, :]

    for j in range(NCHUNK):
        pltpu.sync_copy(grad_v.at[pl.ds(j * CHUNK, CHUNK)],
                        accum_sp.at[rep_v.at[j]], add=True)

    plsc.subcore_barrier()

    for j in range(NCHUNK):
        pltpu.sync_copy(accum_sp.at[rep_v.at[j]],
                        grad_v.at[pl.ds(j * CHUNK, CHUNK)])

    @pl.when(c == 0)
    def _():
        for j in range(NCHUNK):
            pltpu.sync_copy(grad_v.at[pl.ds(j * CHUNK, CHUNK)],
                            di_hbm.at[idxl_v.at[j]])

    @pl.when(c == 1)
    def _():
        for j in range(NCHUNK):
            pltpu.sync_copy(grad_v.at[pl.ds(j * CHUNK, CHUNK)],
                            du_hbm.at[idxl_v.at[j]])


_sc_fixup = pl.kernel(
    _sc_body,
    out_type=(),
    mesh=plsc.VectorSubcoreMesh(core_axis_name="c", subcore_axis_name="s"),
    compiler_params=pltpu.CompilerParams(use_tc_tiling_on_sc=False),
    scratch_types=[
        pltpu.VMEM_SHARED((N_ITEMS,), jnp.int32),
        pltpu.VMEM_SHARED((B_,), jnp.float32),
        pltpu.VMEM_SHARED((B_, DIM), jnp.float32),
        pltpu.VMEM((NCHUNK, CHUNK), jnp.int32),
        pltpu.VMEM((NCHUNK, CHUNK), jnp.int32),
        pltpu.VMEM((NCHUNK, CHUNK), jnp.int32),
        pltpu.VMEM((NCHUNK, CHUNK), jnp.float32),
        pltpu.VMEM((NCHUNK, CHUNK), jnp.float32),
        pltpu.VMEM((CHUNK,), jnp.float32),
        pltpu.VMEM((PER_TILE, DIM), jnp.float32),
    ],
)


def kernel(item_emb, user_emb, item_grad, user_grad, returned_items,
           returned_users):
    ri = returned_items.astype(jnp.int32)
    ru = returned_users.astype(jnp.int32)
    idx_l = jnp.concatenate([ri, ru]).reshape(2 * B_ // CHUNK, CHUNK)
    occ = jnp.arange(B_, dtype=jnp.int32).reshape(B_ // CHUNK, CHUNK)
    grads = jnp.concatenate([item_grad, user_grad], axis=0)
    zeros2 = jnp.zeros((PER_TILE, DIM), jnp.float32)
    zeros1 = jnp.zeros((PER_TILE,), jnp.float32)
    ones1 = jnp.ones((CHUNK,), jnp.float32)

    di_ref = jax.new_ref(jnp.zeros((N_ITEMS, DIM), jnp.float32))
    du_ref = jax.new_ref(jnp.zeros((N_USERS, DIM), jnp.float32))
    _sc_fixup(di_ref, du_ref, idx_l, occ, grads, zeros2, zeros1, ones1)
    out_i = _combine(item_emb.T, di_ref[...].T)
    out_u = _combine(user_emb.T, du_ref[...].T)
    return jnp.concatenate([out_i.T, out_u.T], axis=0)
